# baseline (speedup 1.0000x reference)
import functools

import jax
import jax.numpy as jnp
from jax import lax
from jax.experimental import pallas as pl
from jax.experimental.pallas import tpu as pltpu
from jax.experimental.pallas import tpu_sc as plsc

N_NODES = 10000
N_EDGES = 320000
N_GRAPHS = 128
EMB = 128
NC = 2
NS = 16
CH = 40
EPT = N_EDGES // (NC * NS)
NCHUNK = EPT // CH
NPAD = 10240
RPT = NPAD // NS
CNTW = 16

F32 = jnp.float32


NPIPE = 5
NGROUP = NCHUNK // NPIPE


def _make_sc_agg():
    mesh = plsc.VectorSubcoreMesh(core_axis_name="c", subcore_axis_name="s")
    outs = (
        jax.ShapeDtypeStruct((NPAD, EMB), F32),
        jax.ShapeDtypeStruct((NPAD, EMB), F32),
    )
    scratch = (
        [pltpu.VMEM_SHARED((NPAD, EMB), F32)]
        + [pltpu.VMEM((CH,), jnp.int32) for _ in range(2 * NPIPE)]
        + [pltpu.VMEM((CH, EMB), F32) for _ in range(NPIPE)]
        + [pltpu.SemaphoreType.DMA for _ in range(NPIPE)]
    )

    @functools.partial(pl.kernel, out_type=outs, mesh=mesh, scratch_types=scratch)
    def sc(u, srcr, dstr, zz, outa, outb, accum, *scr):
        sidx = scr[0:NPIPE]
        didx = scr[NPIPE:2 * NPIPE]
        rows = scr[2 * NPIPE:3 * NPIPE]
        sems = scr[3 * NPIPE:]
        c = lax.axis_index("c")
        s = lax.axis_index("s")
        r0 = s * RPT
        pltpu.sync_copy(zz, accum.at[pl.ds(r0, RPT)])
        plsc.subcore_barrier()

        ebase = c * (N_EDGES // NC) + s * EPT

        def step(g, _):
            base = ebase + g * (NPIPE * CH)
            cps = []
            for k in range(NPIPE):
                pltpu.sync_copy(srcr.at[pl.ds(base + k * CH, CH)], sidx[k])
                pltpu.sync_copy(dstr.at[pl.ds(base + k * CH, CH)], didx[k])
                cps.append(pltpu.async_copy(u.at[sidx[k]], rows[k], sems[k]))
            for k in range(NPIPE):
                cps[k].wait()
                pltpu.sync_copy(rows[k], accum.at[didx[k]], add=True)
            return ()

        lax.fori_loop(0, NGROUP, step, ())
        plsc.subcore_barrier()

        @pl.when(c == 0)
        def _():
            pltpu.sync_copy(accum.at[pl.ds(r0, RPT)], outa.at[pl.ds(r0, RPT)])

        @pl.when(c == 1)
        def _():
            pltpu.sync_copy(accum.at[pl.ds(r0, RPT)], outb.at[pl.ds(r0, RPT)])

    return sc


def _make_sc_count():
    mesh = plsc.VectorSubcoreMesh(core_axis_name="c", subcore_axis_name="s")
    outs = (
        jax.ShapeDtypeStruct((NPAD, EMB), F32),
        jax.ShapeDtypeStruct((NPAD, EMB), F32),
    )
    scratch = [
        pltpu.VMEM_SHARED((NPAD, EMB), F32),
        pltpu.VMEM((CH,), jnp.int32),
        pltpu.VMEM((CH, EMB), F32),
    ]

    @functools.partial(pl.kernel, out_type=outs, mesh=mesh, scratch_types=scratch)
    def sc(dstr, onesb, zz, outa, outb, accum, didx, onesv):
        c = lax.axis_index("c")
        s = lax.axis_index("s")
        r0 = s * RPT
        pltpu.sync_copy(zz, accum.at[pl.ds(r0, RPT)])
        pltpu.sync_copy(onesb, onesv)
        plsc.subcore_barrier()

        ebase = c * (N_EDGES // NC) + s * EPT

        def step(i, _):
            off = ebase + i * CH
            pltpu.sync_copy(dstr.at[pl.ds(off, CH)], didx)
            pltpu.sync_copy(onesv, accum.at[didx], add=True)
            return ()

        lax.fori_loop(0, NCHUNK, step, ())
        plsc.subcore_barrier()

        @pl.when(c == 0)
        def _():
            pltpu.sync_copy(accum.at[pl.ds(r0, RPT)], outa.at[pl.ds(r0, RPT)])

        @pl.when(c == 1)
        def _():
            pltpu.sync_copy(accum.at[pl.ds(r0, RPT)], outb.at[pl.ds(r0, RPT)])

    return sc


_sc_agg = _make_sc_agg()
_sc_count = _make_sc_count()


NB = 1000
NGRID = N_NODES // NB


def _enc_body(x0, x1, dep, tp, ap, dp, w_rel, w_root, b1, u_out, r_out):
    oh0 = (x0[...] == lax.broadcasted_iota(jnp.int32, (NB, 128), 1)).astype(F32)
    oh1 = (x1[...] == lax.broadcasted_iota(jnp.int32, (NB, 128), 1)).astype(F32)
    ohd = (dep[...] == lax.broadcasted_iota(jnp.int32, (NB, 32), 1)).astype(F32)
    h = (jnp.dot(oh0, tp[...], preferred_element_type=F32)
         + jnp.dot(oh1, ap[...], preferred_element_type=F32)
         + jnp.dot(ohd, dp[...], preferred_element_type=F32))
    u_out[...] = jnp.dot(h, w_rel[...], preferred_element_type=F32)
    r_out[...] = jnp.dot(h, w_root[...], preferred_element_type=F32) + b1[...]


def _tc_encode(x0, x1, dep, tp, ap, dp, w_rel, w_root, b1):
    return pl.pallas_call(
        _enc_body,
        grid=(NGRID,),
        in_specs=[
            pl.BlockSpec((NB, 1), lambda i: (i, 0)),
            pl.BlockSpec((NB, 1), lambda i: (i, 0)),
            pl.BlockSpec((NB, 1), lambda i: (i, 0)),
            pl.BlockSpec((128, 128), lambda i: (0, 0)),
            pl.BlockSpec((128, 128), lambda i: (0, 0)),
            pl.BlockSpec((32, 128), lambda i: (0, 0)),
            pl.BlockSpec((128, 128), lambda i: (0, 0)),
            pl.BlockSpec((128, 128), lambda i: (0, 0)),
            pl.BlockSpec((1, 128), lambda i: (0, 0)),
        ],
        out_specs=[
            pl.BlockSpec((NB, EMB), lambda i: (i, 0)),
            pl.BlockSpec((NB, EMB), lambda i: (i, 0)),
        ],
        out_shape=[
            jax.ShapeDtypeStruct((N_NODES, EMB), F32),
            jax.ShapeDtypeStruct((N_NODES, EMB), F32),
        ],
    )(x0, x1, dep, tp, ap, dp, w_rel, w_root, b1)


def _mid_body(sa, sb, ca, cb, r, bat, w_rel, w_root, b2,
              u_out, r_out, xs_out, gc_out, cnt_out):
    i = pl.program_id(0)
    cntl = ca[...] + cb[...]
    cnt = jnp.sum(cntl, axis=1, keepdims=True) * (1.0 / CNTW)
    agg = (sa[...] + sb[...]) / jnp.maximum(cnt, 1.0)
    h = jnp.maximum(agg + r[...], 0.0)
    u_out[...] = jnp.dot(h, w_rel[...], preferred_element_type=F32)
    r_out[...] = jnp.dot(h, w_root[...], preferred_element_type=F32) + b2[...]
    cnt_out[...] = cntl
    oh = (bat[...] == lax.broadcasted_iota(jnp.int32, (NB, N_GRAPHS), 1)).astype(F32)
    contrib = lax.dot_general(oh, h, (((0,), (0,)), ((), ())),
                              preferred_element_type=F32)
    gcontrib = lax.dot_general(oh, jnp.ones((NB, 128), F32),
                               (((0,), (0,)), ((), ())),
                               preferred_element_type=F32)

    @pl.when(i == 0)
    def _():
        xs_out[...] = jnp.zeros_like(xs_out)
        gc_out[...] = jnp.zeros_like(gc_out)

    xs_out[...] += contrib
    gc_out[...] += gcontrib


def _tc_mid(sa, sb, ca, cb, r, bat, w_rel, w_root, b2):
    return pl.pallas_call(
        _mid_body,
        grid=(NGRID,),
        in_specs=[
            pl.BlockSpec((NB, EMB), lambda i: (i, 0)),
            pl.BlockSpec((NB, EMB), lambda i: (i, 0)),
            pl.BlockSpec((NB, CNTW), lambda i: (i, 0)),
            pl.BlockSpec((NB, CNTW), lambda i: (i, 0)),
            pl.BlockSpec((NB, EMB), lambda i: (i, 0)),
            pl.BlockSpec((NB, 1), lambda i: (i, 0)),
            pl.BlockSpec((128, 128), lambda i: (0, 0)),
            pl.BlockSpec((128, 128), lambda i: (0, 0)),
            pl.BlockSpec((1, 128), lambda i: (0, 0)),
        ],
        out_specs=[
            pl.BlockSpec((NB, EMB), lambda i: (i, 0)),
            pl.BlockSpec((NB, EMB), lambda i: (i, 0)),
            pl.BlockSpec((N_GRAPHS, EMB), lambda i: (0, 0)),
            pl.BlockSpec((N_GRAPHS, 128), lambda i: (0, 0)),
            pl.BlockSpec((NB, CNTW), lambda i: (i, 0)),
        ],
        out_shape=[
            jax.ShapeDtypeStruct((N_NODES, EMB), F32),
            jax.ShapeDtypeStruct((N_NODES, EMB), F32),
            jax.ShapeDtypeStruct((N_GRAPHS, EMB), F32),
            jax.ShapeDtypeStruct((N_GRAPHS, 128), F32),
            jax.ShapeDtypeStruct((N_NODES, CNTW), F32),
        ],
    )(sa, sb, ca, cb, r, bat, w_rel, w_root, b2)


def _last_body(sa, sb, cn, r, bat, xs_out):
    i = pl.program_id(0)
    cnt = jnp.sum(cn[...], axis=1, keepdims=True) * (1.0 / CNTW)
    agg = (sa[...] + sb[...]) / jnp.maximum(cnt, 1.0)
    h = jnp.maximum(agg + r[...], 0.0)
    oh = (bat[...] == lax.broadcasted_iota(jnp.int32, (NB, N_GRAPHS), 1)).astype(F32)
    contrib = lax.dot_general(oh, h, (((0,), (0,)), ((), ())),
                              preferred_element_type=F32)

    @pl.when(i == 0)
    def _():
        xs_out[...] = jnp.zeros_like(xs_out)

    xs_out[...] += contrib


def _tc_last(sa, sb, cn, r, bat):
    return pl.pallas_call(
        _last_body,
        grid=(NGRID,),
        in_specs=[
            pl.BlockSpec((NB, EMB), lambda i: (i, 0)),
            pl.BlockSpec((NB, EMB), lambda i: (i, 0)),
            pl.BlockSpec((NB, CNTW), lambda i: (i, 0)),
            pl.BlockSpec((NB, EMB), lambda i: (i, 0)),
            pl.BlockSpec((NB, 1), lambda i: (i, 0)),
        ],
        out_specs=[pl.BlockSpec((N_GRAPHS, EMB), lambda i: (0, 0))],
        out_shape=[jax.ShapeDtypeStruct((N_GRAPHS, EMB), F32)],
    )(sa, sb, cn, r, bat)


def _head_body(xs0, xs1, gc, wa, wb, bl, wp, bp, out):
    g = jnp.maximum(gc[...][:, :EMB], 1.0)
    a = xs0[...] / g
    b = xs1[...] / g
    o = jnp.maximum(jnp.dot(a, wa[...], preferred_element_type=F32)
                    + jnp.dot(b, wb[...], preferred_element_type=F32)
                    + bl[...], 0.0)
    out[...] = jnp.dot(o, wp[...], preferred_element_type=F32) + bp[...]


def _tc_head(xs0, xs1, gc, wa, wb, bl, wp, bp):
    return pl.pallas_call(
        _head_body,
        out_shape=jax.ShapeDtypeStruct((N_GRAPHS, 128), F32),
    )(xs0, xs1, gc, wa, wb, bl, wp, bp)


def kernel(x, edge_index, node_depth, batch, type_emb, attr_emb, depth_emb,
           W1_rel, b1_rel, W1_root, W2_rel, b2_rel, W2_root,
           W_lin1, b_lin1, W_pred, b_pred):
    x0 = x[:, 0:1].astype(jnp.int32)
    x1 = x[:, 1:2].astype(jnp.int32)
    dep = jnp.clip(node_depth, 0, 20).astype(jnp.int32)
    bat = batch.reshape(N_NODES, 1).astype(jnp.int32)
    src = edge_index[0].astype(jnp.int32)
    dst = edge_index[1].astype(jnp.int32)

    tp = jnp.pad(type_emb, ((0, 128 - type_emb.shape[0]), (0, 0)))
    ap = jnp.pad(attr_emb, ((0, 128 - attr_emb.shape[0]), (0, 0)))
    dp = jnp.pad(depth_emb, ((0, 32 - depth_emb.shape[0]), (0, 0)))
    b1 = b1_rel.reshape(1, EMB)
    b2 = b2_rel.reshape(1, EMB)
    bl = b_lin1.reshape(1, EMB)
    bp = b_pred.reshape(1, 128)
    wa = W_lin1[:EMB]
    wb = W_lin1[EMB:]

    zz = jnp.zeros((RPT, EMB), F32)
    onesb = jnp.ones((CH, EMB), F32)

    cfa, cfb = _sc_count(dst, onesb, zz)
    ca, cb = cfa[:N_NODES, :CNTW], cfb[:N_NODES, :CNTW]
    u0, r0 = _tc_encode(x0, x1, dep, tp, ap, dp, W1_rel, W1_root, b1)
    s0a, s0b = _sc_agg(u0, src, dst, zz)
    s0a, s0b = s0a[:N_NODES], s0b[:N_NODES]
    u1, r1, xs0s, gc, cn = _tc_mid(s0a, s0b, ca, cb, r0, bat, W2_rel, W2_root, b2)
    s1a, s1b = _sc_agg(u1, src, dst, zz)
    s1a, s1b = s1a[:N_NODES], s1b[:N_NODES]
    (xs1s,) = _tc_last(s1a, s1b, cn, r1, bat)
    return _tc_head(xs0s, xs1s, gc, wa, wb, bl, W_pred, bp)


# speedup vs baseline: 1.3508x; 1.3508x over previous
import functools

import jax
import jax.numpy as jnp
from jax import lax
from jax.experimental import pallas as pl
from jax.experimental.pallas import tpu as pltpu
from jax.experimental.pallas import tpu_sc as plsc

N_NODES = 10000
N_EDGES = 320000
N_GRAPHS = 128
EMB = 128
NC = 2
NS = 16
CH = 40
EPT = N_EDGES // (NC * NS)
NCHUNK = EPT // CH
NPAD = 10240
RPT = NPAD // NS
CNTW = 16

F32 = jnp.float32


NPIPE = 5
NGROUP = NCHUNK // NPIPE


def _make_sc_agg():
    mesh = plsc.VectorSubcoreMesh(core_axis_name="c", subcore_axis_name="s")
    outs = (
        jax.ShapeDtypeStruct((NPAD, EMB), F32),
        jax.ShapeDtypeStruct((NPAD, EMB), F32),
    )
    scratch = (
        [pltpu.VMEM_SHARED((NPAD, EMB), F32)]
        + [pltpu.VMEM((CH,), jnp.int32) for _ in range(2 * NPIPE)]
        + [pltpu.VMEM((CH, EMB), F32) for _ in range(NPIPE)]
        + [pltpu.SemaphoreType.DMA for _ in range(3 * NPIPE)]
    )

    @functools.partial(pl.kernel, out_type=outs, mesh=mesh, scratch_types=scratch)
    def sc(u, srcr, dstr, zz, outa, outb, accum, *scr):
        sidx = scr[0:NPIPE]
        didx = scr[NPIPE:2 * NPIPE]
        rows = scr[2 * NPIPE:3 * NPIPE]
        sems = scr[3 * NPIPE:4 * NPIPE]
        isems = scr[4 * NPIPE:5 * NPIPE]
        dsems = scr[5 * NPIPE:]
        c = lax.axis_index("c")
        s = lax.axis_index("s")
        r0 = s * RPT
        pltpu.sync_copy(zz, accum.at[pl.ds(r0, RPT)])
        plsc.subcore_barrier()

        ebase = c * (N_EDGES // NC) + s * EPT

        def step(g, _):
            base = ebase + g * (NPIPE * CH)
            icps = []
            for k in range(NPIPE):
                icps.append(pltpu.async_copy(
                    srcr.at[pl.ds(base + k * CH, CH)], sidx[k], isems[k]))
                icps.append(pltpu.async_copy(
                    dstr.at[pl.ds(base + k * CH, CH)], didx[k], dsems[k]))
            cps = []
            for k in range(NPIPE):
                icps[2 * k].wait()
                cps.append(pltpu.async_copy(u.at[sidx[k]], rows[k], sems[k]))
            for k in range(NPIPE):
                cps[k].wait()
                icps[2 * k + 1].wait()
                pltpu.sync_copy(rows[k], accum.at[didx[k]], add=True)
            return ()

        lax.fori_loop(0, NGROUP, step, ())
        plsc.subcore_barrier()

        @pl.when(c == 0)
        def _():
            pltpu.sync_copy(accum.at[pl.ds(r0, RPT)], outa.at[pl.ds(r0, RPT)])

        @pl.when(c == 1)
        def _():
            pltpu.sync_copy(accum.at[pl.ds(r0, RPT)], outb.at[pl.ds(r0, RPT)])

    return sc


def _make_sc_count():
    mesh = plsc.VectorSubcoreMesh(core_axis_name="c", subcore_axis_name="s")
    outs = (
        jax.ShapeDtypeStruct((NPAD, EMB), F32),
        jax.ShapeDtypeStruct((NPAD, EMB), F32),
    )
    scratch = [
        pltpu.VMEM_SHARED((NPAD, EMB), F32),
        pltpu.VMEM((CH,), jnp.int32),
        pltpu.VMEM((CH, EMB), F32),
    ]

    @functools.partial(pl.kernel, out_type=outs, mesh=mesh, scratch_types=scratch)
    def sc(dstr, onesb, zz, outa, outb, accum, didx, onesv):
        c = lax.axis_index("c")
        s = lax.axis_index("s")
        r0 = s * RPT
        pltpu.sync_copy(zz, accum.at[pl.ds(r0, RPT)])
        pltpu.sync_copy(onesb, onesv)
        plsc.subcore_barrier()

        ebase = c * (N_EDGES // NC) + s * EPT

        def step(i, _):
            off = ebase + i * CH
            pltpu.sync_copy(dstr.at[pl.ds(off, CH)], didx)
            pltpu.sync_copy(onesv, accum.at[didx], add=True)
            return ()

        lax.fori_loop(0, NCHUNK, step, ())
        plsc.subcore_barrier()

        @pl.when(c == 0)
        def _():
            pltpu.sync_copy(accum.at[pl.ds(r0, RPT)], outa.at[pl.ds(r0, RPT)])

        @pl.when(c == 1)
        def _():
            pltpu.sync_copy(accum.at[pl.ds(r0, RPT)], outb.at[pl.ds(r0, RPT)])

    return sc


_sc_agg = _make_sc_agg()
_sc_count = _make_sc_count()


NB = 1000
NGRID = N_NODES // NB


def _enc_body(x0, x1, dep, tp, ap, dp, w_rel, w_root, b1, u_out, r_out):
    oh0 = (x0[...] == lax.broadcasted_iota(jnp.int32, (NB, 128), 1)).astype(F32)
    oh1 = (x1[...] == lax.broadcasted_iota(jnp.int32, (NB, 128), 1)).astype(F32)
    ohd = (dep[...] == lax.broadcasted_iota(jnp.int32, (NB, 32), 1)).astype(F32)
    h = (jnp.dot(oh0, tp[...], preferred_element_type=F32)
         + jnp.dot(oh1, ap[...], preferred_element_type=F32)
         + jnp.dot(ohd, dp[...], preferred_element_type=F32))
    u_out[...] = jnp.dot(h, w_rel[...], preferred_element_type=F32)
    r_out[...] = jnp.dot(h, w_root[...], preferred_element_type=F32) + b1[...]


def _tc_encode(x0, x1, dep, tp, ap, dp, w_rel, w_root, b1):
    return pl.pallas_call(
        _enc_body,
        grid=(NGRID,),
        in_specs=[
            pl.BlockSpec((NB, 1), lambda i: (i, 0)),
            pl.BlockSpec((NB, 1), lambda i: (i, 0)),
            pl.BlockSpec((NB, 1), lambda i: (i, 0)),
            pl.BlockSpec((128, 128), lambda i: (0, 0)),
            pl.BlockSpec((128, 128), lambda i: (0, 0)),
            pl.BlockSpec((32, 128), lambda i: (0, 0)),
            pl.BlockSpec((128, 128), lambda i: (0, 0)),
            pl.BlockSpec((128, 128), lambda i: (0, 0)),
            pl.BlockSpec((1, 128), lambda i: (0, 0)),
        ],
        out_specs=[
            pl.BlockSpec((NB, EMB), lambda i: (i, 0)),
            pl.BlockSpec((NB, EMB), lambda i: (i, 0)),
        ],
        out_shape=[
            jax.ShapeDtypeStruct((N_NODES, EMB), F32),
            jax.ShapeDtypeStruct((N_NODES, EMB), F32),
        ],
    )(x0, x1, dep, tp, ap, dp, w_rel, w_root, b1)


def _mid_body(sa, sb, ca, cb, r, bat, w_rel, w_root, b2,
              u_out, r_out, xs_out, gc_out, cnt_out):
    i = pl.program_id(0)
    cntl = ca[...] + cb[...]
    cnt = jnp.sum(cntl, axis=1, keepdims=True) * (1.0 / CNTW)
    agg = (sa[...] + sb[...]) / jnp.maximum(cnt, 1.0)
    h = jnp.maximum(agg + r[...], 0.0)
    u_out[...] = jnp.dot(h, w_rel[...], preferred_element_type=F32)
    r_out[...] = jnp.dot(h, w_root[...], preferred_element_type=F32) + b2[...]
    cnt_out[...] = cntl
    oh = (bat[...] == lax.broadcasted_iota(jnp.int32, (NB, N_GRAPHS), 1)).astype(F32)
    contrib = lax.dot_general(oh, h, (((0,), (0,)), ((), ())),
                              preferred_element_type=F32)
    gcontrib = lax.dot_general(oh, jnp.ones((NB, 128), F32),
                               (((0,), (0,)), ((), ())),
                               preferred_element_type=F32)

    @pl.when(i == 0)
    def _():
        xs_out[...] = jnp.zeros_like(xs_out)
        gc_out[...] = jnp.zeros_like(gc_out)

    xs_out[...] += contrib
    gc_out[...] += gcontrib


def _tc_mid(sa, sb, ca, cb, r, bat, w_rel, w_root, b2):
    return pl.pallas_call(
        _mid_body,
        grid=(NGRID,),
        in_specs=[
            pl.BlockSpec((NB, EMB), lambda i: (i, 0)),
            pl.BlockSpec((NB, EMB), lambda i: (i, 0)),
            pl.BlockSpec((NB, CNTW), lambda i: (i, 0)),
            pl.BlockSpec((NB, CNTW), lambda i: (i, 0)),
            pl.BlockSpec((NB, EMB), lambda i: (i, 0)),
            pl.BlockSpec((NB, 1), lambda i: (i, 0)),
            pl.BlockSpec((128, 128), lambda i: (0, 0)),
            pl.BlockSpec((128, 128), lambda i: (0, 0)),
            pl.BlockSpec((1, 128), lambda i: (0, 0)),
        ],
        out_specs=[
            pl.BlockSpec((NB, EMB), lambda i: (i, 0)),
            pl.BlockSpec((NB, EMB), lambda i: (i, 0)),
            pl.BlockSpec((N_GRAPHS, EMB), lambda i: (0, 0)),
            pl.BlockSpec((N_GRAPHS, 128), lambda i: (0, 0)),
            pl.BlockSpec((NB, CNTW), lambda i: (i, 0)),
        ],
        out_shape=[
            jax.ShapeDtypeStruct((N_NODES, EMB), F32),
            jax.ShapeDtypeStruct((N_NODES, EMB), F32),
            jax.ShapeDtypeStruct((N_GRAPHS, EMB), F32),
            jax.ShapeDtypeStruct((N_GRAPHS, 128), F32),
            jax.ShapeDtypeStruct((N_NODES, CNTW), F32),
        ],
    )(sa, sb, ca, cb, r, bat, w_rel, w_root, b2)


def _last_body(sa, sb, cn, r, bat, xs_out):
    i = pl.program_id(0)
    cnt = jnp.sum(cn[...], axis=1, keepdims=True) * (1.0 / CNTW)
    agg = (sa[...] + sb[...]) / jnp.maximum(cnt, 1.0)
    h = jnp.maximum(agg + r[...], 0.0)
    oh = (bat[...] == lax.broadcasted_iota(jnp.int32, (NB, N_GRAPHS), 1)).astype(F32)
    contrib = lax.dot_general(oh, h, (((0,), (0,)), ((), ())),
                              preferred_element_type=F32)

    @pl.when(i == 0)
    def _():
        xs_out[...] = jnp.zeros_like(xs_out)

    xs_out[...] += contrib


def _tc_last(sa, sb, cn, r, bat):
    return pl.pallas_call(
        _last_body,
        grid=(NGRID,),
        in_specs=[
            pl.BlockSpec((NB, EMB), lambda i: (i, 0)),
            pl.BlockSpec((NB, EMB), lambda i: (i, 0)),
            pl.BlockSpec((NB, CNTW), lambda i: (i, 0)),
            pl.BlockSpec((NB, EMB), lambda i: (i, 0)),
            pl.BlockSpec((NB, 1), lambda i: (i, 0)),
        ],
        out_specs=[pl.BlockSpec((N_GRAPHS, EMB), lambda i: (0, 0))],
        out_shape=[jax.ShapeDtypeStruct((N_GRAPHS, EMB), F32)],
    )(sa, sb, cn, r, bat)


def _head_body(xs0, xs1, gc, wa, wb, bl, wp, bp, out):
    g = jnp.maximum(gc[...][:, :EMB], 1.0)
    a = xs0[...] / g
    b = xs1[...] / g
    o = jnp.maximum(jnp.dot(a, wa[...], preferred_element_type=F32)
                    + jnp.dot(b, wb[...], preferred_element_type=F32)
                    + bl[...], 0.0)
    out[...] = jnp.dot(o, wp[...], preferred_element_type=F32) + bp[...]


def _tc_head(xs0, xs1, gc, wa, wb, bl, wp, bp):
    return pl.pallas_call(
        _head_body,
        out_shape=jax.ShapeDtypeStruct((N_GRAPHS, 128), F32),
    )(xs0, xs1, gc, wa, wb, bl, wp, bp)


def kernel(x, edge_index, node_depth, batch, type_emb, attr_emb, depth_emb,
           W1_rel, b1_rel, W1_root, W2_rel, b2_rel, W2_root,
           W_lin1, b_lin1, W_pred, b_pred):
    x0 = x[:, 0:1].astype(jnp.int32)
    x1 = x[:, 1:2].astype(jnp.int32)
    dep = jnp.clip(node_depth, 0, 20).astype(jnp.int32)
    bat = batch.reshape(N_NODES, 1).astype(jnp.int32)
    src = edge_index[0].astype(jnp.int32)
    dst = edge_index[1].astype(jnp.int32)

    tp = jnp.pad(type_emb, ((0, 128 - type_emb.shape[0]), (0, 0)))
    ap = jnp.pad(attr_emb, ((0, 128 - attr_emb.shape[0]), (0, 0)))
    dp = jnp.pad(depth_emb, ((0, 32 - depth_emb.shape[0]), (0, 0)))
    b1 = b1_rel.reshape(1, EMB)
    b2 = b2_rel.reshape(1, EMB)
    bl = b_lin1.reshape(1, EMB)
    bp = b_pred.reshape(1, 128)
    wa = W_lin1[:EMB]
    wb = W_lin1[EMB:]

    zz = jnp.zeros((RPT, EMB), F32)
    onesb = jnp.ones((CH, EMB), F32)

    cfa, cfb = _sc_count(dst, onesb, zz)
    ca, cb = cfa[:N_NODES, :CNTW], cfb[:N_NODES, :CNTW]
    u0, r0 = _tc_encode(x0, x1, dep, tp, ap, dp, W1_rel, W1_root, b1)
    s0a, s0b = _sc_agg(u0, src, dst, zz)
    s0a, s0b = s0a[:N_NODES], s0b[:N_NODES]
    u1, r1, xs0s, gc, cn = _tc_mid(s0a, s0b, ca, cb, r0, bat, W2_rel, W2_root, b2)
    s1a, s1b = _sc_agg(u1, src, dst, zz)
    s1a, s1b = s1a[:N_NODES], s1b[:N_NODES]
    (xs1s,) = _tc_last(s1a, s1b, cn, r1, bat)
    return _tc_head(xs0s, xs1s, gc, wa, wb, bl, W_pred, bp)


# speedup vs baseline: 1.5427x; 1.1421x over previous
import functools

import jax
import jax.numpy as jnp
from jax import lax
from jax.experimental import pallas as pl
from jax.experimental.pallas import tpu as pltpu
from jax.experimental.pallas import tpu_sc as plsc

N_NODES = 10000
N_EDGES = 320000
N_GRAPHS = 128
EMB = 128
NC = 2
NS = 16
CH = 40
EPT = N_EDGES // (NC * NS)
NCHUNK = EPT // CH
NPAD = 10240
RPT = NPAD // NS
CNTW = 16

F32 = jnp.float32


NPIPE = 5
NGROUP = NCHUNK // NPIPE


def _make_sc_agg():
    mesh = plsc.VectorSubcoreMesh(core_axis_name="c", subcore_axis_name="s")
    outs = (
        jax.ShapeDtypeStruct((NPAD, EMB), F32),
        jax.ShapeDtypeStruct((NPAD, EMB), F32),
    )
    scratch = (
        [pltpu.VMEM_SHARED((NPAD, EMB), F32)]
        + [pltpu.VMEM((CH,), jnp.int32) for _ in range(2 * NPIPE)]
        + [pltpu.VMEM((CH, EMB), F32) for _ in range(NPIPE)]
        + [pltpu.SemaphoreType.DMA for _ in range(3 * NPIPE)]
    )

    @functools.partial(pl.kernel, out_type=outs, mesh=mesh, scratch_types=scratch)
    def sc(u, srcr, dstr, zz, outa, outb, accum, *scr):
        sidx = scr[0:NPIPE]
        didx = scr[NPIPE:2 * NPIPE]
        rows = scr[2 * NPIPE:3 * NPIPE]
        sems = scr[3 * NPIPE:4 * NPIPE]
        isems = scr[4 * NPIPE:5 * NPIPE]
        dsems = scr[5 * NPIPE:]
        c = lax.axis_index("c")
        s = lax.axis_index("s")
        r0 = s * RPT
        pltpu.sync_copy(zz, accum.at[pl.ds(r0, RPT)])
        plsc.subcore_barrier()

        ebase = c * (N_EDGES // NC) + s * EPT

        def step(g, _):
            base = ebase + g * (NPIPE * CH)
            icps = []
            for k in range(NPIPE):
                icps.append(pltpu.async_copy(
                    srcr.at[pl.ds(base + k * CH, CH)], sidx[k], isems[k]))
                icps.append(pltpu.async_copy(
                    dstr.at[pl.ds(base + k * CH, CH)], didx[k], dsems[k]))
            cps = []
            for k in range(NPIPE):
                icps[2 * k].wait()
                cps.append(pltpu.async_copy(u.at[sidx[k]], rows[k], sems[k]))
            for k in range(NPIPE):
                cps[k].wait()
                icps[2 * k + 1].wait()
                pltpu.sync_copy(rows[k], accum.at[didx[k]], add=True)
            return ()

        lax.fori_loop(0, NGROUP, step, ())
        plsc.subcore_barrier()

        @pl.when(c == 0)
        def _():
            pltpu.sync_copy(accum.at[pl.ds(r0, RPT)], outa.at[pl.ds(r0, RPT)])

        @pl.when(c == 1)
        def _():
            pltpu.sync_copy(accum.at[pl.ds(r0, RPT)], outb.at[pl.ds(r0, RPT)])

    return sc


def _make_sc_count():
    mesh = plsc.VectorSubcoreMesh(core_axis_name="c", subcore_axis_name="s")
    outs = (
        jax.ShapeDtypeStruct((NPAD, EMB), F32),
        jax.ShapeDtypeStruct((NPAD, EMB), F32),
    )
    scratch = (
        [pltpu.VMEM_SHARED((NPAD, EMB), F32),
         pltpu.VMEM((CH, EMB), F32)]
        + [pltpu.VMEM((CH,), jnp.int32) for _ in range(NPIPE)]
        + [pltpu.SemaphoreType.DMA for _ in range(NPIPE)]
    )

    @functools.partial(pl.kernel, out_type=outs, mesh=mesh, scratch_types=scratch)
    def sc(dstr, onesb, zz, outa, outb, accum, onesv, *scr):
        didx = scr[0:NPIPE]
        dsems = scr[NPIPE:]
        c = lax.axis_index("c")
        s = lax.axis_index("s")
        r0 = s * RPT
        pltpu.sync_copy(zz, accum.at[pl.ds(r0, RPT)])
        pltpu.sync_copy(onesb, onesv)
        plsc.subcore_barrier()

        ebase = c * (N_EDGES // NC) + s * EPT

        def step(g, _):
            base = ebase + g * (NPIPE * CH)
            icps = []
            for k in range(NPIPE):
                icps.append(pltpu.async_copy(
                    dstr.at[pl.ds(base + k * CH, CH)], didx[k], dsems[k]))
            for k in range(NPIPE):
                icps[k].wait()
                pltpu.sync_copy(onesv, accum.at[didx[k]], add=True)
            return ()

        lax.fori_loop(0, NGROUP, step, ())
        plsc.subcore_barrier()

        @pl.when(c == 0)
        def _():
            pltpu.sync_copy(accum.at[pl.ds(r0, RPT)], outa.at[pl.ds(r0, RPT)])

        @pl.when(c == 1)
        def _():
            pltpu.sync_copy(accum.at[pl.ds(r0, RPT)], outb.at[pl.ds(r0, RPT)])

    return sc


_sc_agg = _make_sc_agg()
_sc_count = _make_sc_count()


NB = 1000
NGRID = N_NODES // NB


def _enc_body(x0, x1, dep, tp, ap, dp, w_rel, w_root, b1, u_out, r_out):
    oh0 = (x0[...] == lax.broadcasted_iota(jnp.int32, (NB, 128), 1)).astype(F32)
    oh1 = (x1[...] == lax.broadcasted_iota(jnp.int32, (NB, 128), 1)).astype(F32)
    ohd = (dep[...] == lax.broadcasted_iota(jnp.int32, (NB, 32), 1)).astype(F32)
    h = (jnp.dot(oh0, tp[...], preferred_element_type=F32)
         + jnp.dot(oh1, ap[...], preferred_element_type=F32)
         + jnp.dot(ohd, dp[...], preferred_element_type=F32))
    u_out[...] = jnp.dot(h, w_rel[...], preferred_element_type=F32)
    r_out[...] = jnp.dot(h, w_root[...], preferred_element_type=F32) + b1[...]


def _tc_encode(x0, x1, dep, tp, ap, dp, w_rel, w_root, b1):
    return pl.pallas_call(
        _enc_body,
        grid=(NGRID,),
        in_specs=[
            pl.BlockSpec((NB, 1), lambda i: (i, 0)),
            pl.BlockSpec((NB, 1), lambda i: (i, 0)),
            pl.BlockSpec((NB, 1), lambda i: (i, 0)),
            pl.BlockSpec((128, 128), lambda i: (0, 0)),
            pl.BlockSpec((128, 128), lambda i: (0, 0)),
            pl.BlockSpec((32, 128), lambda i: (0, 0)),
            pl.BlockSpec((128, 128), lambda i: (0, 0)),
            pl.BlockSpec((128, 128), lambda i: (0, 0)),
            pl.BlockSpec((1, 128), lambda i: (0, 0)),
        ],
        out_specs=[
            pl.BlockSpec((NB, EMB), lambda i: (i, 0)),
            pl.BlockSpec((NB, EMB), lambda i: (i, 0)),
        ],
        out_shape=[
            jax.ShapeDtypeStruct((N_NODES, EMB), F32),
            jax.ShapeDtypeStruct((N_NODES, EMB), F32),
        ],
    )(x0, x1, dep, tp, ap, dp, w_rel, w_root, b1)


def _mid_body(sa, sb, ca, cb, r, bat, w_rel, w_root, b2,
              u_out, r_out, xs_out, gc_out, cnt_out):
    i = pl.program_id(0)
    cntl = ca[...] + cb[...]
    cnt = jnp.sum(cntl, axis=1, keepdims=True) * (1.0 / CNTW)
    agg = (sa[...] + sb[...]) / jnp.maximum(cnt, 1.0)
    h = jnp.maximum(agg + r[...], 0.0)
    u_out[...] = jnp.dot(h, w_rel[...], preferred_element_type=F32)
    r_out[...] = jnp.dot(h, w_root[...], preferred_element_type=F32) + b2[...]
    cnt_out[...] = cntl
    oh = (bat[...] == lax.broadcasted_iota(jnp.int32, (NB, N_GRAPHS), 1)).astype(F32)
    contrib = lax.dot_general(oh, h, (((0,), (0,)), ((), ())),
                              preferred_element_type=F32)
    gcontrib = lax.dot_general(oh, jnp.ones((NB, 128), F32),
                               (((0,), (0,)), ((), ())),
                               preferred_element_type=F32)

    @pl.when(i == 0)
    def _():
        xs_out[...] = jnp.zeros_like(xs_out)
        gc_out[...] = jnp.zeros_like(gc_out)

    xs_out[...] += contrib
    gc_out[...] += gcontrib


def _tc_mid(sa, sb, ca, cb, r, bat, w_rel, w_root, b2):
    return pl.pallas_call(
        _mid_body,
        grid=(NGRID,),
        in_specs=[
            pl.BlockSpec((NB, EMB), lambda i: (i, 0)),
            pl.BlockSpec((NB, EMB), lambda i: (i, 0)),
            pl.BlockSpec((NB, CNTW), lambda i: (i, 0)),
            pl.BlockSpec((NB, CNTW), lambda i: (i, 0)),
            pl.BlockSpec((NB, EMB), lambda i: (i, 0)),
            pl.BlockSpec((NB, 1), lambda i: (i, 0)),
            pl.BlockSpec((128, 128), lambda i: (0, 0)),
            pl.BlockSpec((128, 128), lambda i: (0, 0)),
            pl.BlockSpec((1, 128), lambda i: (0, 0)),
        ],
        out_specs=[
            pl.BlockSpec((NB, EMB), lambda i: (i, 0)),
            pl.BlockSpec((NB, EMB), lambda i: (i, 0)),
            pl.BlockSpec((N_GRAPHS, EMB), lambda i: (0, 0)),
            pl.BlockSpec((N_GRAPHS, 128), lambda i: (0, 0)),
            pl.BlockSpec((NB, CNTW), lambda i: (i, 0)),
        ],
        out_shape=[
            jax.ShapeDtypeStruct((N_NODES, EMB), F32),
            jax.ShapeDtypeStruct((N_NODES, EMB), F32),
            jax.ShapeDtypeStruct((N_GRAPHS, EMB), F32),
            jax.ShapeDtypeStruct((N_GRAPHS, 128), F32),
            jax.ShapeDtypeStruct((N_NODES, CNTW), F32),
        ],
    )(sa, sb, ca, cb, r, bat, w_rel, w_root, b2)


def _last_body(sa, sb, cn, r, bat, xs_out):
    i = pl.program_id(0)
    cnt = jnp.sum(cn[...], axis=1, keepdims=True) * (1.0 / CNTW)
    agg = (sa[...] + sb[...]) / jnp.maximum(cnt, 1.0)
    h = jnp.maximum(agg + r[...], 0.0)
    oh = (bat[...] == lax.broadcasted_iota(jnp.int32, (NB, N_GRAPHS), 1)).astype(F32)
    contrib = lax.dot_general(oh, h, (((0,), (0,)), ((), ())),
                              preferred_element_type=F32)

    @pl.when(i == 0)
    def _():
        xs_out[...] = jnp.zeros_like(xs_out)

    xs_out[...] += contrib


def _tc_last(sa, sb, cn, r, bat):
    return pl.pallas_call(
        _last_body,
        grid=(NGRID,),
        in_specs=[
            pl.BlockSpec((NB, EMB), lambda i: (i, 0)),
            pl.BlockSpec((NB, EMB), lambda i: (i, 0)),
            pl.BlockSpec((NB, CNTW), lambda i: (i, 0)),
            pl.BlockSpec((NB, EMB), lambda i: (i, 0)),
            pl.BlockSpec((NB, 1), lambda i: (i, 0)),
        ],
        out_specs=[pl.BlockSpec((N_GRAPHS, EMB), lambda i: (0, 0))],
        out_shape=[jax.ShapeDtypeStruct((N_GRAPHS, EMB), F32)],
    )(sa, sb, cn, r, bat)


def _head_body(xs0, xs1, gc, wa, wb, bl, wp, bp, out):
    g = jnp.maximum(gc[...][:, :EMB], 1.0)
    a = xs0[...] / g
    b = xs1[...] / g
    o = jnp.maximum(jnp.dot(a, wa[...], preferred_element_type=F32)
                    + jnp.dot(b, wb[...], preferred_element_type=F32)
                    + bl[...], 0.0)
    out[...] = jnp.dot(o, wp[...], preferred_element_type=F32) + bp[...]


def _tc_head(xs0, xs1, gc, wa, wb, bl, wp, bp):
    return pl.pallas_call(
        _head_body,
        out_shape=jax.ShapeDtypeStruct((N_GRAPHS, 128), F32),
    )(xs0, xs1, gc, wa, wb, bl, wp, bp)


def kernel(x, edge_index, node_depth, batch, type_emb, attr_emb, depth_emb,
           W1_rel, b1_rel, W1_root, W2_rel, b2_rel, W2_root,
           W_lin1, b_lin1, W_pred, b_pred):
    x0 = x[:, 0:1].astype(jnp.int32)
    x1 = x[:, 1:2].astype(jnp.int32)
    dep = jnp.clip(node_depth, 0, 20).astype(jnp.int32)
    bat = batch.reshape(N_NODES, 1).astype(jnp.int32)
    src = edge_index[0].astype(jnp.int32)
    dst = edge_index[1].astype(jnp.int32)

    tp = jnp.pad(type_emb, ((0, 128 - type_emb.shape[0]), (0, 0)))
    ap = jnp.pad(attr_emb, ((0, 128 - attr_emb.shape[0]), (0, 0)))
    dp = jnp.pad(depth_emb, ((0, 32 - depth_emb.shape[0]), (0, 0)))
    b1 = b1_rel.reshape(1, EMB)
    b2 = b2_rel.reshape(1, EMB)
    bl = b_lin1.reshape(1, EMB)
    bp = b_pred.reshape(1, 128)
    wa = W_lin1[:EMB]
    wb = W_lin1[EMB:]

    zz = jnp.zeros((RPT, EMB), F32)
    onesb = jnp.ones((CH, EMB), F32)

    cfa, cfb = _sc_count(dst, onesb, zz)
    ca, cb = cfa[:N_NODES, :CNTW], cfb[:N_NODES, :CNTW]
    u0, r0 = _tc_encode(x0, x1, dep, tp, ap, dp, W1_rel, W1_root, b1)
    s0a, s0b = _sc_agg(u0, src, dst, zz)
    s0a, s0b = s0a[:N_NODES], s0b[:N_NODES]
    u1, r1, xs0s, gc, cn = _tc_mid(s0a, s0b, ca, cb, r0, bat, W2_rel, W2_root, b2)
    s1a, s1b = _sc_agg(u1, src, dst, zz)
    s1a, s1b = s1a[:N_NODES], s1b[:N_NODES]
    (xs1s,) = _tc_last(s1a, s1b, cn, r1, bat)
    return _tc_head(xs0s, xs1s, gc, wa, wb, bl, W_pred, bp)
